# baseline (speedup 1.0000x reference)
import functools

import jax
import jax.numpy as jnp
import numpy as np
from jax import lax
from jax.experimental import pallas as pl
from jax.experimental.pallas import tpu as pltpu
from jax.experimental.pallas import tpu_sc as plsc

RADIUS = 0.2
NSAMPLE = 32
POS_DIM = 24
EPS = 1e-5

N_PTS = 8192
S_CTR = 1024
SBLK = 256
CHUNK = 256
D_PACK = 128
GCHUNK = 512
RBLK = 1024


def _ballq_body(ct_ref, xyzt_ref, idx_ref, gidx_ref, cnt_scr, slots_scr):
    b = pl.program_id(0)
    n = N_PTS
    cnt_scr[...] = jnp.zeros((SBLK, 1), jnp.float32)
    slots_scr[...] = jnp.full((SBLK, NSAMPLE), float(n), jnp.float32)

    ct = ct_ref[0]
    cx = ct[:, 0:1]
    cy = ct[:, 1:2]
    cz = ct[:, 2:3]
    c2 = cx * cx + cy * cy + cz * cz

    ri = lax.broadcasted_iota(jnp.int32, (CHUNK, CHUNK), 0)
    ci = lax.broadcasted_iota(jnp.int32, (CHUNK, CHUNK), 1)
    tri = (ri <= ci).astype(jnp.float32)

    r2 = jnp.float32(RADIUS ** 2)

    def body(c, carry):
        xx = xyzt_ref[0, 0:1, pl.ds(c * CHUNK, CHUNK)]
        xy = xyzt_ref[0, 1:2, pl.ds(c * CHUNK, CHUNK)]
        xz = xyzt_ref[0, 2:3, pl.ds(c * CHUNK, CHUNK)]
        x2 = xx * xx + xy * xy + xz * xz
        rb = lambda v: v.astype(jnp.bfloat16).astype(jnp.float32)
        dot = rb(cx) * rb(xx) + rb(cy) * rb(xy) + rb(cz) * rb(xz)
        d = (-2.0 * dot + c2) + x2
        mask = d <= r2
        maskf = mask.astype(jnp.float32)

        @pl.when(jnp.min(cnt_scr[...]) < float(NSAMPLE))
        def _():
            incl = jax.lax.dot(maskf, tri,
                               preferred_element_type=jnp.float32)
            rank = cnt_scr[...] + incl - maskf
            rmask = jnp.where(mask, rank, -1.0)
            idxrow = jnp.float32(c * CHUNK) + lax.broadcasted_iota(
                jnp.int32, (SBLK, CHUNK), 1).astype(jnp.float32)
            for k in range(NSAMPLE):
                cand = jnp.where(rmask == float(k), idxrow, float(n))
                m = jnp.min(cand, axis=1, keepdims=True)
                slots_scr[:, k:k + 1] = jnp.minimum(slots_scr[:, k:k + 1], m)
            cnt_scr[...] = cnt_scr[...] + jnp.sum(maskf, axis=1,
                                                  keepdims=True)
        return carry

    lax.fori_loop(0, n // CHUNK, body, 0)

    slots = slots_scr[...]
    first = slots[:, 0:1]
    idx = jnp.where(slots == float(n), jnp.broadcast_to(first, slots.shape),
                    slots)
    idx_ref[0] = idx.astype(jnp.int32)
    gidx_ref[0] = (jnp.minimum(idx, float(n - 1))
                   + jnp.float32(n) * b.astype(jnp.float32)).astype(jnp.int32)


def _ball_query(patch_center, xyzt):
    B = patch_center.shape[0]
    grid = (B, S_CTR // SBLK)
    return pl.pallas_call(
        _ballq_body,
        grid=grid,
        in_specs=[
            pl.BlockSpec((1, SBLK, 3), lambda b, s: (b, s, 0)),
            pl.BlockSpec((1, 3, N_PTS), lambda b, s: (b, 0, 0)),
        ],
        out_specs=[
            pl.BlockSpec((1, SBLK, NSAMPLE), lambda b, s: (b, s, 0)),
            pl.BlockSpec((1, SBLK, NSAMPLE), lambda b, s: (b, s, 0)),
        ],
        out_shape=[
            jax.ShapeDtypeStruct((B, S_CTR, NSAMPLE), jnp.int32),
            jax.ShapeDtypeStruct((B, S_CTR, NSAMPLE), jnp.int32),
        ],
        scratch_shapes=[
            pltpu.VMEM((SBLK, 1), jnp.float32),
            pltpu.VMEM((SBLK, NSAMPLE), jnp.float32),
        ],
    )(patch_center, xyzt)


def _sc_gather(table, flat_idx):
    rows = flat_idx.shape[0]
    mesh = plsc.VectorSubcoreMesh(core_axis_name="c", subcore_axis_name="s")
    nw = 32
    per_w = rows // nw
    n_chunks = per_w // GCHUNK

    @functools.partial(
        pl.kernel,
        mesh=mesh,
        out_type=jax.ShapeDtypeStruct((rows, D_PACK), jnp.float32),
        scratch_types=[
            pltpu.VMEM((GCHUNK,), jnp.int32),
            pltpu.VMEM((GCHUNK, D_PACK), jnp.float32),
            pltpu.SemaphoreType.DMA,
        ],
    )
    def gather_k(table_hbm, idx_hbm, out_hbm, idx_v, rows_v, sem):
        wid = lax.axis_index("s") * 2 + lax.axis_index("c")
        base = wid * per_w

        def body(i, carry):
            off = base + i * GCHUNK
            pltpu.sync_copy(idx_hbm.at[pl.ds(off, GCHUNK)], idx_v)
            pltpu.async_copy(table_hbm.at[idx_v], rows_v, sem).wait()
            pltpu.sync_copy(rows_v, out_hbm.at[pl.ds(off, GCHUNK)])
            return carry

        lax.fori_loop(0, n_chunks, body, 0)

    return gather_k(table, flat_idx)


def _mlp_body(g_ref, cb_ref, w1f_ref, w1x_ref, w1p_ref, b1_ref, g1_ref,
              be1_ref, w2_ref, b2_ref, g2_ref, be2_ref, out_ref):
    g = g_ref[...]
    c = cb_ref[...]
    rel = g[:, 0:3] - c
    feat = g[:, 3:67]

    nf = POS_DIM // 6
    pow2 = jnp.left_shift(jnp.int32(1),
                          lax.broadcasted_iota(jnp.int32, (1, nf), 1))
    freqs = pow2.astype(jnp.float32) * jnp.float32(np.pi)
    pieces = []
    for ci in range(3):
        sc = rel[:, ci:ci + 1] * freqs
        pieces.append(jnp.sin(sc))
        pieces.append(jnp.cos(sc))
    pos = jnp.concatenate(pieces, axis=1)

    h = (jax.lax.dot(feat, w1f_ref[...], preferred_element_type=jnp.float32)
         + jax.lax.dot(rel, w1x_ref[...], preferred_element_type=jnp.float32)
         + jax.lax.dot(pos, w1p_ref[...], preferred_element_type=jnp.float32)
         + b1_ref[...])
    mu = jnp.mean(h, axis=1, keepdims=True)
    dlt = h - mu
    var = jnp.mean(dlt * dlt, axis=1, keepdims=True)
    h = dlt / jnp.sqrt(var + EPS) * g1_ref[...] + be1_ref[...]
    y = jnp.float32(np.sqrt(2.0 / np.pi)) * (
        h + jnp.float32(0.044715) * (h * h * h))
    a = jnp.exp(-2.0 * jnp.abs(y))
    t = (1.0 - a) / (1.0 + a)
    t = jnp.where(y < 0.0, -t, t)
    h = 0.5 * h * (1.0 + t)

    h = jax.lax.dot(h, w2_ref[...],
                    preferred_element_type=jnp.float32) + b2_ref[...]
    mu = jnp.mean(h, axis=1, keepdims=True)
    dlt = h - mu
    var = jnp.mean(dlt * dlt, axis=1, keepdims=True)
    h = dlt / jnp.sqrt(var + EPS) * g2_ref[...] + be2_ref[...]

    hr = h.reshape(RBLK // NSAMPLE, NSAMPLE, 128)
    out_ref[...] = jnp.max(hr, axis=1)


def _mlp_pool(G, CB, W1f, W1x, W1p, b1, g1, be1, W2, b2, g2, be2):
    rows = G.shape[0]
    grid = (rows // RBLK,)
    rep2 = lambda shape: pl.BlockSpec(shape, lambda i: (0, 0))
    return pl.pallas_call(
        _mlp_body,
        grid=grid,
        in_specs=[
            pl.BlockSpec((RBLK, D_PACK), lambda i: (i, 0)),
            pl.BlockSpec((RBLK, 3), lambda i: (i, 0)),
            rep2((64, 128)), rep2((3, 128)), rep2((24, 128)),
            rep2((1, 128)), rep2((1, 128)), rep2((1, 128)),
            rep2((128, 128)), rep2((1, 128)), rep2((1, 128)),
            rep2((1, 128)),
        ],
        out_specs=pl.BlockSpec((RBLK // NSAMPLE, 128), lambda i: (i, 0)),
        out_shape=jax.ShapeDtypeStruct((rows // NSAMPLE, 128), jnp.float32),
    )(G, CB, W1f, W1x, W1p, b1, g1, be1, W2, b2, g2, be2)


def kernel(xyz, point_feature, patch_center, W1, b1, g1, be1, W2, b2, g2,
           be2):
    B, N, _ = xyz.shape
    S = patch_center.shape[1]
    stem = point_feature.shape[-1]

    xyzt = jnp.transpose(xyz, (0, 2, 1))
    neighbor_idx, gidx = _ball_query(patch_center, xyzt)

    pad = D_PACK - 3 - stem
    table = jnp.concatenate(
        [xyz, point_feature,
         jnp.zeros((B, N, pad), jnp.float32)], axis=-1).reshape(B * N, D_PACK)
    G = _sc_gather(table, gidx.reshape(-1))

    CB = jnp.broadcast_to(patch_center[:, :, None, :],
                          (B, S, NSAMPLE, 3)).reshape(-1, 3)

    W1f = W1[0:stem]
    W1x = W1[stem:stem + 3]
    W1p = W1[stem + 3:]
    r1 = lambda v: v.reshape(1, -1)
    pf = _mlp_pool(G, CB, W1f, W1x, W1p, r1(b1), r1(g1), r1(be1), W2,
                   r1(b2), r1(g2), r1(be2))
    return pf.reshape(B, S, 128), neighbor_idx


# speedup vs baseline: 1.5519x; 1.5519x over previous
import functools

import jax
import jax.numpy as jnp
import numpy as np
from jax import lax
from jax.experimental import pallas as pl
from jax.experimental.pallas import tpu as pltpu
from jax.experimental.pallas import tpu_sc as plsc

RADIUS = 0.2
NSAMPLE = 32
POS_DIM = 24
EPS = 1e-5

N_PTS = 8192
S_CTR = 1024
SBLK = 256
CHUNK = 256
D_PACK = 128
GCHUNK = 512
RBLK = 1024


def _ballq_body(ct_ref, xyzt_ref, idx_ref, gidx_ref, cnt_scr, slots_scr):
    b = pl.program_id(0)
    n = N_PTS
    cnt_scr[...] = jnp.zeros((SBLK, 1), jnp.float32)
    slots_scr[...] = jnp.full((SBLK, NSAMPLE), float(n), jnp.float32)

    ct = ct_ref[0]
    cx = ct[:, 0:1]
    cy = ct[:, 1:2]
    cz = ct[:, 2:3]
    c2 = cx * cx + cy * cy + cz * cz

    ri = lax.broadcasted_iota(jnp.int32, (CHUNK, CHUNK), 0)
    ci = lax.broadcasted_iota(jnp.int32, (CHUNK, CHUNK), 1)
    tri = (ri <= ci).astype(jnp.float32)

    r2 = jnp.float32(RADIUS ** 2)

    def body(c, carry):
        xx = xyzt_ref[0, 0:1, pl.ds(c * CHUNK, CHUNK)]
        xy = xyzt_ref[0, 1:2, pl.ds(c * CHUNK, CHUNK)]
        xz = xyzt_ref[0, 2:3, pl.ds(c * CHUNK, CHUNK)]
        x2 = xx * xx + xy * xy + xz * xz
        rb = lambda v: v.astype(jnp.bfloat16).astype(jnp.float32)
        dot = rb(cx) * rb(xx) + rb(cy) * rb(xy) + rb(cz) * rb(xz)
        d = (-2.0 * dot + c2) + x2
        mask = d <= r2
        maskf = mask.astype(jnp.float32)

        @pl.when(jnp.min(cnt_scr[...]) < float(NSAMPLE))
        def _():
            incl = jax.lax.dot(maskf, tri,
                               preferred_element_type=jnp.float32)
            rank = cnt_scr[...] + incl - maskf
            rmask = jnp.where(mask, rank, -1.0)
            idxrow = jnp.float32(c * CHUNK) + lax.broadcasted_iota(
                jnp.int32, (SBLK, CHUNK), 1).astype(jnp.float32)
            for k in range(NSAMPLE):
                cand = jnp.where(rmask == float(k), idxrow, float(n))
                m = jnp.min(cand, axis=1, keepdims=True)
                slots_scr[:, k:k + 1] = jnp.minimum(slots_scr[:, k:k + 1], m)
            cnt_scr[...] = cnt_scr[...] + jnp.sum(maskf, axis=1,
                                                  keepdims=True)
        return carry

    lax.fori_loop(0, n // CHUNK, body, 0)

    slots = slots_scr[...]
    first = slots[:, 0:1]
    idx = jnp.where(slots == float(n), jnp.broadcast_to(first, slots.shape),
                    slots)
    idx_ref[0] = idx.astype(jnp.int32)
    gidx_ref[0] = (jnp.minimum(idx, float(n - 1))
                   + jnp.float32(n) * b.astype(jnp.float32)).astype(jnp.int32)


def _ball_query(patch_center, xyzt):
    B = patch_center.shape[0]
    grid = (B, S_CTR // SBLK)
    return pl.pallas_call(
        _ballq_body,
        grid=grid,
        in_specs=[
            pl.BlockSpec((1, SBLK, 3), lambda b, s: (b, s, 0)),
            pl.BlockSpec((1, 3, N_PTS), lambda b, s: (b, 0, 0)),
        ],
        out_specs=[
            pl.BlockSpec((1, SBLK, NSAMPLE), lambda b, s: (b, s, 0)),
            pl.BlockSpec((1, SBLK, NSAMPLE), lambda b, s: (b, s, 0)),
        ],
        out_shape=[
            jax.ShapeDtypeStruct((B, S_CTR, NSAMPLE), jnp.int32),
            jax.ShapeDtypeStruct((B, S_CTR, NSAMPLE), jnp.int32),
        ],
        scratch_shapes=[
            pltpu.VMEM((SBLK, 1), jnp.float32),
            pltpu.VMEM((SBLK, NSAMPLE), jnp.float32),
        ],
    )(patch_center, xyzt)


def _sc_gather(table, flat_idx):
    rows = flat_idx.shape[0]
    mesh = plsc.VectorSubcoreMesh(core_axis_name="c", subcore_axis_name="s")
    nw = 32
    per_w = rows // nw
    n_chunks = per_w // GCHUNK

    @functools.partial(
        pl.kernel,
        mesh=mesh,
        out_type=jax.ShapeDtypeStruct((rows, D_PACK), jnp.float32),
        scratch_types=[
            pltpu.VMEM((GCHUNK,), jnp.int32),
            pltpu.VMEM((GCHUNK, D_PACK), jnp.float32),
            pltpu.SemaphoreType.DMA,
        ],
    )
    def gather_k(table_hbm, idx_hbm, out_hbm, idx_v, rows_v, sem):
        wid = lax.axis_index("s") * 2 + lax.axis_index("c")
        base = wid * per_w

        def body(i, carry):
            off = base + i * GCHUNK
            pltpu.sync_copy(idx_hbm.at[pl.ds(off, GCHUNK)], idx_v)
            pltpu.async_copy(table_hbm.at[idx_v], rows_v, sem).wait()
            pltpu.sync_copy(rows_v, out_hbm.at[pl.ds(off, GCHUNK)])
            return carry

        lax.fori_loop(0, n_chunks, body, 0)

    return gather_k(table, flat_idx)


def _mlp_body(g_ref, cb_ref, w1f_ref, w1x_ref, w1p_ref, b1_ref, g1_ref,
              be1_ref, w2_ref, b2_ref, g2_ref, be2_ref, out_ref):
    g = g_ref[...]
    c = cb_ref[...]
    rel = g[:, 0:3] - c
    feat = g[:, 3:67]

    nf = POS_DIM // 6
    pow2 = jnp.left_shift(jnp.int32(1),
                          lax.broadcasted_iota(jnp.int32, (1, nf), 1))
    freqs = pow2.astype(jnp.float32) * jnp.float32(np.pi)
    halfpi = jnp.float32(np.pi / 2.0)
    phases = []
    for ci in range(3):
        sc = rel[:, ci:ci + 1] * freqs
        phases.append(sc)
        phases.append(sc + halfpi)
    ph = jnp.concatenate(phases, axis=1)
    q = jnp.floor(ph * jnp.float32(1.0 / np.pi) + 0.5)
    r = (ph - q * jnp.float32(3.140625)) - q * jnp.float32(9.67653589793e-4)
    parity = q - 2.0 * jnp.floor(q * 0.5)
    sgn = 1.0 - 2.0 * parity
    r2 = r * r
    p = jnp.float32(2.7557319e-6)
    p = p * r2 + jnp.float32(-1.9841270e-4)
    p = p * r2 + jnp.float32(8.3333333e-3)
    p = p * r2 + jnp.float32(-1.6666667e-1)
    p = p * r2 * r + r
    pos = sgn * p

    h = (jax.lax.dot(feat, w1f_ref[...], preferred_element_type=jnp.float32)
         + jax.lax.dot(rel, w1x_ref[...], preferred_element_type=jnp.float32)
         + jax.lax.dot(pos, w1p_ref[...], preferred_element_type=jnp.float32)
         + b1_ref[...])
    mu = jnp.mean(h, axis=1, keepdims=True)
    dlt = h - mu
    var = jnp.mean(dlt * dlt, axis=1, keepdims=True)
    h = dlt / jnp.sqrt(var + EPS) * g1_ref[...] + be1_ref[...]
    y = jnp.float32(np.sqrt(2.0 / np.pi)) * (
        h + jnp.float32(0.044715) * (h * h * h))
    a = jnp.exp(-2.0 * jnp.abs(y))
    t = (1.0 - a) / (1.0 + a)
    t = jnp.where(y < 0.0, -t, t)
    h = 0.5 * h * (1.0 + t)

    h = jax.lax.dot(h, w2_ref[...],
                    preferred_element_type=jnp.float32) + b2_ref[...]
    mu = jnp.mean(h, axis=1, keepdims=True)
    dlt = h - mu
    var = jnp.mean(dlt * dlt, axis=1, keepdims=True)
    h = dlt / jnp.sqrt(var + EPS) * g2_ref[...] + be2_ref[...]

    hr = h.reshape(RBLK // NSAMPLE, NSAMPLE, 128)
    out_ref[...] = jnp.max(hr, axis=1)


def _mlp_pool(G, CB, W1f, W1x, W1p, b1, g1, be1, W2, b2, g2, be2):
    rows = G.shape[0]
    grid = (rows // RBLK,)
    rep2 = lambda shape: pl.BlockSpec(shape, lambda i: (0, 0))
    return pl.pallas_call(
        _mlp_body,
        grid=grid,
        in_specs=[
            pl.BlockSpec((RBLK, D_PACK), lambda i: (i, 0)),
            pl.BlockSpec((RBLK, 3), lambda i: (i, 0)),
            rep2((64, 128)), rep2((3, 128)), rep2((24, 128)),
            rep2((1, 128)), rep2((1, 128)), rep2((1, 128)),
            rep2((128, 128)), rep2((1, 128)), rep2((1, 128)),
            rep2((1, 128)),
        ],
        out_specs=pl.BlockSpec((RBLK // NSAMPLE, 128), lambda i: (i, 0)),
        out_shape=jax.ShapeDtypeStruct((rows // NSAMPLE, 128), jnp.float32),
    )(G, CB, W1f, W1x, W1p, b1, g1, be1, W2, b2, g2, be2)


def kernel(xyz, point_feature, patch_center, W1, b1, g1, be1, W2, b2, g2,
           be2):
    B, N, _ = xyz.shape
    S = patch_center.shape[1]
    stem = point_feature.shape[-1]

    xyzt = jnp.transpose(xyz, (0, 2, 1))
    neighbor_idx, gidx = _ball_query(patch_center, xyzt)

    pad = D_PACK - 3 - stem
    table = jnp.concatenate(
        [xyz, point_feature,
         jnp.zeros((B, N, pad), jnp.float32)], axis=-1).reshape(B * N, D_PACK)
    G = _sc_gather(table, gidx.reshape(-1))

    CB = jnp.broadcast_to(patch_center[:, :, None, :],
                          (B, S, NSAMPLE, 3)).reshape(-1, 3)

    W1f = W1[0:stem]
    W1x = W1[stem:stem + 3]
    W1p = W1[stem + 3:]
    r1 = lambda v: v.reshape(1, -1)
    pf = _mlp_pool(G, CB, W1f, W1x, W1p, r1(b1), r1(g1), r1(be1), W2,
                   r1(b2), r1(g2), r1(be2))
    return pf.reshape(B, S, 128), neighbor_idx
